# Initial kernel scaffold; baseline (speedup 1.0000x reference)
import jax
import jax.numpy as jnp
from jax.experimental import pallas as pl


def kernel(t, idx):
    raise NotImplementedError("write your pallas kernel here")


# speedup vs baseline: 1.2491x; 1.2491x over previous
import functools

import jax
import jax.numpy as jnp
from jax import lax
from jax.experimental import pallas as pl
from jax.experimental.pallas import tpu as pltpu
from jax.experimental.pallas import tpu_sc as plsc

_A = 4
_N = 1024
_C = 16
_K = 8192
_D = _A * _C

_NC = 2
_NS = 16
_NW = _NC * _NS

_GA_CH = 16
_GA_W = 128

_GB_CH = 2
_GB_W = 128

_MESH = plsc.VectorSubcoreMesh(core_axis_name="c", subcore_axis_name="s")
_SC_PARAMS = pltpu.CompilerParams(use_tc_tiling_on_sc=False)


def _worker_id():
    return lax.axis_index("s") * _NC + lax.axis_index("c")


@functools.partial(
    pl.kernel,
    out_type=jax.ShapeDtypeStruct((_NW, _GA_CH, _GA_W), jnp.float32),
    mesh=_MESH,
    compiler_params=_SC_PARAMS,
    scratch_types=[
        pltpu.VMEM((_GA_CH, _GA_W), jnp.int32),
        pltpu.VMEM((_GA_CH, _GA_W), jnp.float32),
        pltpu.SemaphoreType.DMA,
    ],
)
def _diag_extract(t_flat, off_hbm, d_out, off_v, val_v, sem):
    wid = _worker_id()
    pltpu.sync_copy(off_hbm.at[wid], off_v)
    copies = [
        pltpu.async_copy(t_flat.at[off_v.at[j]], val_v.at[j], sem)
        for j in range(_GA_CH)
    ]
    for cp in copies:
        cp.wait()
    pltpu.sync_copy(val_v, d_out.at[wid])


@functools.partial(
    pl.kernel,
    out_type=jax.ShapeDtypeStruct((_NW, _GB_CH, _GB_W, _D), jnp.float32),
    mesh=_MESH,
    compiler_params=_SC_PARAMS,
    scratch_types=[
        pltpu.VMEM((_GB_CH, _GB_W), jnp.int32),
        pltpu.VMEM((_GB_CH, _GB_W, _D), jnp.float32),
        pltpu.SemaphoreType.DMA,
    ],
)
def _row_gather(d_hbm, idx_hbm, out_hbm, idx_v, rows_v, sem):
    wid = _worker_id()
    pltpu.sync_copy(idx_hbm.at[wid], idx_v)
    copies = [
        pltpu.async_copy(d_hbm.at[idx_v.at[j]], rows_v.at[j], sem)
        for j in range(_GB_CH)
    ]
    for cp in copies:
        cp.wait()
    pltpu.sync_copy(rows_v, out_hbm.at[wid])


def kernel(t, idx):
    t_flat = t.reshape(-1)
    g = jnp.arange(_N * _D, dtype=jnp.int32)
    i = g >> 6
    a = (g >> 4) & 3
    c = g & 15
    off = a * (_N * _C * _N) + i * (_C * _N + 1) + c * _N
    off = off.astype(jnp.int32).reshape(_NW, _GA_CH, _GA_W)
    d = _diag_extract(t_flat, off)
    d = d.reshape(_N, _D)
    idx2 = idx.astype(jnp.int32).reshape(_NW, _GB_CH, _GB_W)
    out = _row_gather(d, idx2)
    return out.reshape(_K, _A, _C)


# speedup vs baseline: 5.9275x; 4.7454x over previous
import functools

import jax
import jax.numpy as jnp
from jax import lax
from jax.experimental import pallas as pl
from jax.experimental.pallas import tpu as pltpu
from jax.experimental.pallas import tpu_sc as plsc

_A = 4
_N = 1024
_C = 16
_K = 8192
_D = _A * _C

_IB = 128
_NBLK = _N // _IB

_NC = 2
_NS = 16
_NW = _NC * _NS

_GB_CH = 2
_GB_W = 128


def _diag_block(t_ref, d_ref):
    m = (lax.broadcasted_iota(jnp.int32, (_IB, 1, _IB), 0) ==
         lax.broadcasted_iota(jnp.int32, (_IB, 1, _IB), 2)).astype(jnp.float32)
    parts = []
    for a in range(_A):
        parts.append(jnp.sum(t_ref[a] * m, axis=-1))
    d_ref[...] = jnp.concatenate(parts, axis=-1)


def _diag_extract(t):
    return pl.pallas_call(
        _diag_block,
        grid=(_NBLK,),
        in_specs=[pl.BlockSpec((_A, _IB, _C, _IB), lambda ib: (0, ib, 0, ib))],
        out_specs=pl.BlockSpec((_IB, _D), lambda ib: (ib, 0)),
        out_shape=jax.ShapeDtypeStruct((_N, _D), jnp.float32),
    )(t)


@functools.partial(
    pl.kernel,
    out_type=jax.ShapeDtypeStruct((_NW, _GB_CH, _GB_W, _D), jnp.float32),
    mesh=plsc.VectorSubcoreMesh(core_axis_name="c", subcore_axis_name="s"),
    compiler_params=pltpu.CompilerParams(use_tc_tiling_on_sc=False),
    scratch_types=[
        pltpu.VMEM((_GB_CH, _GB_W), jnp.int32),
        pltpu.VMEM((_GB_CH, _GB_W, _D), jnp.float32),
        pltpu.SemaphoreType.DMA,
    ],
)
def _row_gather(d_hbm, idx_hbm, out_hbm, idx_v, rows_v, sem):
    wid = lax.axis_index("s") * _NC + lax.axis_index("c")
    pltpu.sync_copy(idx_hbm.at[wid], idx_v)
    copies = [
        pltpu.async_copy(d_hbm.at[idx_v.at[j]], rows_v.at[j], sem)
        for j in range(_GB_CH)
    ]
    for cp in copies:
        cp.wait()
    pltpu.sync_copy(rows_v, out_hbm.at[wid])


def kernel(t, idx):
    d = _diag_extract(t)
    idx2 = idx.astype(jnp.int32).reshape(_NW, _GB_CH, _GB_W)
    out = _row_gather(d, idx2)
    return out.reshape(_K, _A, _C)


# speedup vs baseline: 6.4760x; 1.0925x over previous
import functools

import jax
import jax.numpy as jnp
from jax import lax
from jax.experimental import pallas as pl
from jax.experimental.pallas import tpu as pltpu
from jax.experimental.pallas import tpu_sc as plsc

_A = 4
_N = 1024
_C = 16
_K = 8192
_D = _A * _C

_IB = 128
_NBLK = _N // _IB

_NC = 2
_NS = 16
_NW = _NC * _NS

_GB_CH = 2
_GB_W = 128


def _diag_block(t_ref, d_ref):
    m = (lax.broadcasted_iota(jnp.int32, (_IB, 1, _IB), 0) ==
         lax.broadcasted_iota(jnp.int32, (_IB, 1, _IB), 2)).astype(jnp.float32)
    parts = []
    for a in range(_A):
        parts.append(jnp.sum(t_ref[a] * m, axis=-1))
    d_ref[...] = jnp.concatenate(parts, axis=-1)


def _diag_extract(t):
    return pl.pallas_call(
        _diag_block,
        grid=(_NBLK,),
        in_specs=[pl.BlockSpec((_A, _IB, _C, _IB), lambda ib: (0, ib, 0, ib))],
        out_specs=pl.BlockSpec((_IB, _D), lambda ib: (ib, 0)),
        out_shape=jax.ShapeDtypeStruct((_N, _D), jnp.float32),
    )(t)


@functools.partial(
    pl.kernel,
    out_type=jax.ShapeDtypeStruct((_K, _D), jnp.float32),
    mesh=plsc.VectorSubcoreMesh(core_axis_name="c", subcore_axis_name="s"),
    compiler_params=pltpu.CompilerParams(use_tc_tiling_on_sc=False),
    scratch_types=[
        pltpu.VMEM((_GB_CH, _GB_W), jnp.int32),
        pltpu.VMEM((_GB_CH, _GB_W, _D), jnp.float32),
        pltpu.SemaphoreType.DMA,
    ],
)
def _row_gather(d_hbm, idx_hbm, out_hbm, idx_v, rows_v, sem):
    wid = lax.axis_index("s") * _NC + lax.axis_index("c")
    base = wid * _GB_CH * _GB_W
    for j in range(_GB_CH):
        pltpu.sync_copy(idx_hbm.at[pl.ds(base + j * _GB_W, _GB_W)],
                        idx_v.at[j])
    copies = [
        pltpu.async_copy(d_hbm.at[idx_v.at[j]], rows_v.at[j], sem)
        for j in range(_GB_CH)
    ]
    for cp in copies:
        cp.wait()
    for j in range(_GB_CH):
        pltpu.sync_copy(rows_v.at[j],
                        out_hbm.at[pl.ds(base + j * _GB_W, _GB_W)])


def kernel(t, idx):
    d = _diag_extract(t)
    out = _row_gather(d, idx.astype(jnp.int32))
    return out.reshape(_K, _A, _C)
